# baseline (speedup 1.0000x reference)
import functools

import jax
import jax.numpy as jnp
from jax import lax
from jax.experimental import pallas as pl
from jax.experimental.pallas import tpu as pltpu
from jax.experimental.pallas import tpu_sc as plsc

NUM_ENT = 1000000
NUM_REL = 500
DIM = 64
B = 16384
MARGIN = 1.0
REG_W = 0.01

NUM_WORKERS = 32
BPW = B // NUM_WORKERS

ENT_BLK = 8000
GRID = NUM_ENT // ENT_BLK
TRIP_BLK = 256
TRIP_STEPS = B // TRIP_BLK


def _sc_gather_body(ent_hbm, rel_hbm, h_hbm, r_hbm, t_hbm, tc_hbm,
                    hg_hbm, rg_hbm, tg_hbm, tcg_hbm,
                    idx_v, rows_v, sem):
    wid = lax.axis_index("s") * 2 + lax.axis_index("c")
    base = wid * BPW
    for idx_hbm, table, out in (
        (h_hbm, ent_hbm, hg_hbm),
        (r_hbm, rel_hbm, rg_hbm),
        (t_hbm, ent_hbm, tg_hbm),
        (tc_hbm, ent_hbm, tcg_hbm),
    ):
        pltpu.sync_copy(idx_hbm.at[pl.ds(base, BPW)], idx_v)
        pltpu.async_copy(table.at[idx_v], rows_v, sem).wait()
        pltpu.sync_copy(rows_v, out.at[pl.ds(base, BPW)])


_sc_gather = functools.partial(
    pl.kernel,
    mesh=plsc.VectorSubcoreMesh(core_axis_name="c", subcore_axis_name="s"),
    compiler_params=pltpu.CompilerParams(use_tc_tiling_on_sc=False),
    out_type=[jax.ShapeDtypeStruct((B, DIM), jnp.float32)] * 4,
    scratch_types=[
        pltpu.VMEM((BPW,), jnp.int32),
        pltpu.VMEM((BPW, DIM), jnp.float32),
        pltpu.SemaphoreType.DMA,
    ],
)(_sc_gather_body)


def _normalize(x):
    n = jnp.sqrt(jnp.sum(x * x, axis=1, keepdims=True))
    return x / jnp.maximum(n, 1e-12)


def _tc_body(ent_ref, rel_ref, hg_ref, rg_ref, tg_ref, tcg_ref,
             out_ref, acc_ref):
    i = pl.program_id(0)

    @pl.when(i == 0)
    def _init():
        acc_ref[0] = 0.0
        acc_ref[1] = 0.0
        rel = rel_ref[...]
        acc_ref[2] = jnp.sum(rel * rel)

    x = ent_ref[...]
    s = jnp.sum(x * x, axis=1, keepdims=True)
    d = jnp.maximum(jnp.sqrt(s), 1e-12)
    contrib = s / (d * d)
    acc_ref[0] += jnp.sum(contrib)

    @pl.when(i < TRIP_STEPS)
    def _triplets():
        hn = _normalize(hg_ref[...])
        tn = _normalize(tg_ref[...])
        tcn = _normalize(tcg_ref[...])
        r = rg_ref[...]
        pd = hn + r - tn
        nd = hn + r - tcn
        ps = -jnp.sqrt(jnp.sum(pd * pd, axis=1) + 1e-12)
        ns = -jnp.sqrt(jnp.sum(nd * nd, axis=1) + 1e-12)
        hinge = jnp.maximum(0.0, ns - ps + MARGIN)
        acc_ref[1] += jnp.sum(hinge)

    @pl.when(i == GRID - 1)
    def _final():
        out_ref[0] = acc_ref[1] / B + REG_W * (acc_ref[0] + acc_ref[2])


def kernel(entity_table, relation_table, triplets, corrupted_triplets):
    heads = triplets[:, 0]
    rels = triplets[:, 1]
    tails = triplets[:, 2]
    ctails = corrupted_triplets[:, 2]

    hg, rg, tg, tcg = _sc_gather(entity_table, relation_table,
                                 heads, rels, tails, ctails)

    rel_pad = jnp.pad(relation_table, ((0, 512 - NUM_REL), (0, 0)))

    trip_spec = pl.BlockSpec(
        (TRIP_BLK, DIM), lambda i: (jnp.minimum(i, TRIP_STEPS - 1), 0))

    out = pl.pallas_call(
        _tc_body,
        grid=(GRID,),
        in_specs=[
            pl.BlockSpec((ENT_BLK, DIM), lambda i: (i, 0)),
            pl.BlockSpec((512, DIM), lambda i: (0, 0)),
            trip_spec, trip_spec, trip_spec, trip_spec,
        ],
        out_specs=pl.BlockSpec(memory_space=pltpu.SMEM),
        out_shape=jax.ShapeDtypeStruct((1,), jnp.float32),
        scratch_shapes=[pltpu.SMEM((3,), jnp.float32)],
    )(entity_table, rel_pad, hg, rg, tg, tcg)

    return out[0]


# speedup vs baseline: 1.1444x; 1.1444x over previous
import functools

import jax
import jax.numpy as jnp
from jax import lax
from jax.experimental import pallas as pl
from jax.experimental.pallas import tpu as pltpu
from jax.experimental.pallas import tpu_sc as plsc

NUM_ENT = 1000000
NUM_REL = 500
DIM = 64
B = 16384
MARGIN = 1.0
REG_W = 0.01

NUM_WORKERS = 32
BPW = B // NUM_WORKERS

ENT_BLK = 40000
GRID = NUM_ENT // ENT_BLK
TRIP_BLK = 1024
TRIP_STEPS = B // TRIP_BLK


def _sc_gather_body(ent_hbm, rel_hbm, h_hbm, r_hbm, t_hbm, tc_hbm,
                    hg_hbm, rg_hbm, tg_hbm, tcg_hbm,
                    i0, i1, i2, i3, rows0, rows1,
                    gsem, wsem0, wsem1):
    wid = lax.axis_index("s") * 2 + lax.axis_index("c")
    base = wid * BPW
    sl = pl.ds(base, BPW)
    plan = (
        (i0, h_hbm, ent_hbm, hg_hbm),
        (i1, r_hbm, rel_hbm, rg_hbm),
        (i2, t_hbm, ent_hbm, tg_hbm),
        (i3, tc_hbm, ent_hbm, tcg_hbm),
    )
    for idx_v, idx_hbm, _, _ in plan:
        pltpu.sync_copy(idx_hbm.at[sl], idx_v)
    rows = (rows0, rows1)
    wsems = (wsem0, wsem1)
    writes = [None, None]
    gathers = [None] * 4
    gathers[0] = pltpu.async_copy(plan[0][2].at[i0], rows0, gsem)
    for k in range(4):
        gathers[k].wait()
        if k + 1 < 4:
            if writes[(k + 1) % 2] is not None:
                writes[(k + 1) % 2].wait()
            nxt = plan[k + 1]
            gathers[k + 1] = pltpu.async_copy(nxt[2].at[nxt[0]],
                                              rows[(k + 1) % 2], gsem)
        writes[k % 2] = pltpu.async_copy(rows[k % 2], plan[k][3].at[sl],
                                         wsems[k % 2])
    writes[0].wait()
    writes[1].wait()


@functools.cache
def _sc_gather():
    return pl.kernel(
        _sc_gather_body,
        mesh=plsc.VectorSubcoreMesh(core_axis_name="c", subcore_axis_name="s"),
        compiler_params=pltpu.CompilerParams(use_tc_tiling_on_sc=False),
        out_type=[jax.ShapeDtypeStruct((B, DIM), jnp.float32)] * 4,
        scratch_types=[
            pltpu.VMEM((BPW,), jnp.int32),
            pltpu.VMEM((BPW,), jnp.int32),
            pltpu.VMEM((BPW,), jnp.int32),
            pltpu.VMEM((BPW,), jnp.int32),
            pltpu.VMEM((BPW, DIM), jnp.float32),
            pltpu.VMEM((BPW, DIM), jnp.float32),
            pltpu.SemaphoreType.DMA,
            pltpu.SemaphoreType.DMA,
            pltpu.SemaphoreType.DMA,
        ],
    )


def _row_dots(a, b, ones_row):
    return lax.dot_general(ones_row, a * b, (((1,), (1,)), ((), ())),
                           preferred_element_type=jnp.float32)


def _tc_body(ent_ref, rel_ref, hg_ref, rg_ref, tg_ref, tcg_ref,
             out_ref, acc_ref):
    i = pl.program_id(0)

    @pl.when(i == 0)
    def _init():
        acc_ref[0] = 0.0
        acc_ref[1] = 0.0
        rel = rel_ref[...]
        acc_ref[2] = jnp.sum(rel * rel)

    ones_ent = jnp.ones((1, DIM), jnp.float32)

    x = ent_ref[...]
    s = _row_dots(x, x, ones_ent)
    contrib = jnp.minimum(s * 1e24, 1.0)
    acc_ref[0] += jnp.sum(contrib)

    @pl.when(i < TRIP_STEPS)
    def _triplets():
        h = hg_ref[...]
        r = rg_ref[...]
        t = tg_ref[...]
        c = tcg_ref[...]
        hh = _row_dots(h, h, ones_ent)
        tt = _row_dots(t, t, ones_ent)
        cc = _row_dots(c, c, ones_ent)
        rr = _row_dots(r, r, ones_ent)
        hr = _row_dots(h, r, ones_ent)
        ht = _row_dots(h, t, ones_ent)
        hc = _row_dots(h, c, ones_ent)
        rt = _row_dots(r, t, ones_ent)
        rc = _row_dots(r, c, ones_ent)
        dh = jnp.maximum(jnp.sqrt(hh), 1e-12)
        dt = jnp.maximum(jnp.sqrt(tt), 1e-12)
        dc = jnp.maximum(jnp.sqrt(cc), 1e-12)
        nh = hh / (dh * dh)
        pos2 = (nh + tt / (dt * dt) + rr
                + 2.0 * (hr / dh - ht / (dh * dt) - rt / dt))
        neg2 = (nh + cc / (dc * dc) + rr
                + 2.0 * (hr / dh - hc / (dh * dc) - rc / dc))
        ps = -jnp.sqrt(jnp.maximum(pos2, 0.0) + 1e-12)
        ns = -jnp.sqrt(jnp.maximum(neg2, 0.0) + 1e-12)
        hinge = jnp.maximum(0.0, ns - ps + MARGIN)
        acc_ref[1] += jnp.sum(hinge)

    @pl.when(i == GRID - 1)
    def _final():
        out_ref[0] = acc_ref[1] / B + REG_W * (acc_ref[0] + acc_ref[2])


def kernel(entity_table, relation_table, triplets, corrupted_triplets):
    heads = triplets[:, 0]
    rels = triplets[:, 1]
    tails = triplets[:, 2]
    ctails = corrupted_triplets[:, 2]

    hg, rg, tg, tcg = _sc_gather()(entity_table, relation_table,
                                   heads, rels, tails, ctails)

    rel_pad = jnp.pad(relation_table, ((0, 512 - NUM_REL), (0, 0)))

    trip_spec = pl.BlockSpec(
        (TRIP_BLK, DIM), lambda i: (jnp.minimum(i, TRIP_STEPS - 1), 0))

    out = pl.pallas_call(
        _tc_body,
        grid=(GRID,),
        in_specs=[
            pl.BlockSpec((ENT_BLK, DIM), lambda i: (i, 0)),
            pl.BlockSpec((512, DIM), lambda i: (0, 0)),
            trip_spec, trip_spec, trip_spec, trip_spec,
        ],
        out_specs=pl.BlockSpec(memory_space=pltpu.SMEM),
        out_shape=jax.ShapeDtypeStruct((1,), jnp.float32),
        scratch_shapes=[pltpu.SMEM((3,), jnp.float32)],
    )(entity_table, rel_pad, hg, rg, tg, tcg)

    return out[0]


# speedup vs baseline: 3.3450x; 2.9230x over previous
import functools

import jax
import jax.numpy as jnp
from jax import lax
from jax.experimental import pallas as pl
from jax.experimental.pallas import tpu as pltpu
from jax.experimental.pallas import tpu_sc as plsc

NUM_ENT = 1000000
NUM_REL = 500
DIM = 64
B = 16384
MARGIN = 1.0
REG_W = 0.01

NUM_PAIR = NUM_ENT // 2

NUM_WORKERS = 32
BPW = B // NUM_WORKERS
CHUNK = BPW // 2

ENT_BLK = 16384
GRID = 62
OVERLAP = GRID * ENT_BLK - NUM_ENT
PAIR_BLK = ENT_BLK // 2

TRIP_BLK = 1024
TRIP_GRID = B // TRIP_BLK


def _scan_repack_body(ent_ref, rel_ref, pair_ref, stats_ref):
    i = pl.program_id(0)
    ones_row = jnp.ones((1, DIM), jnp.float32)

    @pl.when(i == 0)
    def _init():
        rel = rel_ref[...]
        stats_ref[0] = 0.0
        stats_ref[1] = jnp.sum(rel * rel)

    x = ent_ref[...]
    s = lax.dot_general(ones_row, x * x, (((1,), (0,)), ((), ())),
                        preferred_element_type=jnp.float32)
    contrib = jnp.minimum(s * 1e24, 1.0)
    lane = lax.broadcasted_iota(jnp.int32, (1, ENT_BLK), 1)
    fresh = jnp.logical_or(i < GRID - 1, lane >= OVERLAP)
    stats_ref[0] += jnp.sum(jnp.where(fresh, contrib, 0.0))

    xt0 = jnp.transpose(x[:, :PAIR_BLK])
    xt1 = jnp.transpose(x[:, PAIR_BLK:])
    pair_ref[...] = jnp.concatenate([xt0, xt1], axis=1)


def _scan_repack(ent_t, rel_pair):
    return pl.pallas_call(
        _scan_repack_body,
        grid=(GRID,),
        in_specs=[
            pl.BlockSpec((DIM, ENT_BLK), lambda i: (0, i)),
            pl.BlockSpec((NUM_REL // 2, 2 * DIM), lambda i: (0, 0)),
        ],
        out_specs=[
            pl.BlockSpec((PAIR_BLK, 2 * DIM), lambda i: (i, 0)),
            pl.BlockSpec(memory_space=pltpu.SMEM),
        ],
        out_shape=[
            jax.ShapeDtypeStruct((NUM_PAIR, 2 * DIM), jnp.float32),
            jax.ShapeDtypeStruct((2,), jnp.float32),
        ],
    )(ent_t, rel_pair)


def _sc_gather_body(pair_hbm, relp_hbm, h_hbm, r_hbm, t_hbm, tc_hbm,
                    hg_hbm, rg_hbm, tg_hbm, tcg_hbm,
                    i0, i1, i2, i3, rows0, rows1, gsem, wsem0, wsem1):
    wid = lax.axis_index("s") * 2 + lax.axis_index("c")
    base = wid * BPW
    plan = (
        (i0, h_hbm, pair_hbm, hg_hbm),
        (i1, r_hbm, relp_hbm, rg_hbm),
        (i2, t_hbm, pair_hbm, tg_hbm),
        (i3, tc_hbm, pair_hbm, tcg_hbm),
    )
    for idx_v, idx_hbm, _, _ in plan:
        pltpu.sync_copy(idx_hbm.at[pl.ds(base, BPW)], idx_v)
    rows = (rows0, rows1)
    wsems = (wsem0, wsem1)
    writes = [None, None]
    steps = [(k, half) for k in range(4) for half in range(2)]
    gathers = [None] * 8
    gathers[0] = pltpu.async_copy(
        plan[0][2].at[i0.at[pl.ds(0, CHUNK)]], rows0, gsem)
    for n, (k, half) in enumerate(steps):
        gathers[n].wait()
        if n + 1 < 8:
            k2, h2 = steps[n + 1]
            buf = (n + 1) % 2
            if writes[buf] is not None:
                writes[buf].wait()
            nxt = plan[k2]
            gathers[n + 1] = pltpu.async_copy(
                nxt[2].at[nxt[0].at[pl.ds(h2 * CHUNK, CHUNK)]],
                rows[buf], gsem)
        writes[n % 2] = pltpu.async_copy(
            rows[n % 2], plan[k][3].at[pl.ds(base + half * CHUNK, CHUNK)],
            wsems[n % 2])
    writes[0].wait()
    writes[1].wait()


@functools.cache
def _sc_gather():
    return pl.kernel(
        _sc_gather_body,
        mesh=plsc.VectorSubcoreMesh(core_axis_name="c", subcore_axis_name="s"),
        out_type=[jax.ShapeDtypeStruct((B, 2 * DIM), jnp.float32)] * 4,
        scratch_types=[
            pltpu.VMEM((BPW,), jnp.int32),
            pltpu.VMEM((BPW,), jnp.int32),
            pltpu.VMEM((BPW,), jnp.int32),
            pltpu.VMEM((BPW,), jnp.int32),
            pltpu.VMEM((CHUNK, 2 * DIM), jnp.float32),
            pltpu.VMEM((CHUNK, 2 * DIM), jnp.float32),
            pltpu.SemaphoreType.DMA,
            pltpu.SemaphoreType.DMA,
            pltpu.SemaphoreType.DMA,
        ],
    )


def _row_dots(a, b, ones_row):
    return lax.dot_general(ones_row, a * b, (((1,), (1,)), ((), ())),
                           preferred_element_type=jnp.float32)


def _score_body(hg_ref, rg_ref, tg_ref, tcg_ref, par_ref, stats_ref,
                out_ref, acc_ref):
    i = pl.program_id(0)

    @pl.when(i == 0)
    def _init():
        acc_ref[0] = 0.0

    par = jnp.transpose(par_ref[...])

    def pick(ref, col):
        blk = ref[...]
        even = blk[:, :DIM]
        odd = blk[:, DIM:]
        p = par[:, col:col + 1]
        return even + p * (odd - even)

    h = pick(hg_ref, 0)
    r = pick(rg_ref, 1)
    t = pick(tg_ref, 2)
    c = pick(tcg_ref, 3)

    ones_row = jnp.ones((1, DIM), jnp.float32)
    hh = _row_dots(h, h, ones_row)
    tt = _row_dots(t, t, ones_row)
    cc = _row_dots(c, c, ones_row)
    rr = _row_dots(r, r, ones_row)
    hr = _row_dots(h, r, ones_row)
    ht = _row_dots(h, t, ones_row)
    hc = _row_dots(h, c, ones_row)
    rt = _row_dots(r, t, ones_row)
    rc = _row_dots(r, c, ones_row)
    dh = jnp.maximum(jnp.sqrt(hh), 1e-12)
    dt = jnp.maximum(jnp.sqrt(tt), 1e-12)
    dc = jnp.maximum(jnp.sqrt(cc), 1e-12)
    nh = hh / (dh * dh)
    pos2 = (nh + tt / (dt * dt) + rr
            + 2.0 * (hr / dh - ht / (dh * dt) - rt / dt))
    neg2 = (nh + cc / (dc * dc) + rr
            + 2.0 * (hr / dh - hc / (dh * dc) - rc / dc))
    ps = -jnp.sqrt(jnp.maximum(pos2, 0.0) + 1e-12)
    ns = -jnp.sqrt(jnp.maximum(neg2, 0.0) + 1e-12)
    hinge = jnp.maximum(0.0, ns - ps + MARGIN)
    acc_ref[0] += jnp.sum(hinge)

    @pl.when(i == TRIP_GRID - 1)
    def _final():
        out_ref[0] = acc_ref[0] / B + REG_W * (stats_ref[0] + stats_ref[1])


def _score(hg, rg, tg, tcg, par, stats, interpret=False):
    gspec = pl.BlockSpec((TRIP_BLK, 2 * DIM), lambda i: (i, 0))
    return pl.pallas_call(
        _score_body,
        grid=(TRIP_GRID,),
        in_specs=[
            gspec, gspec, gspec, gspec,
            pl.BlockSpec((4, TRIP_BLK), lambda i: (0, i)),
            pl.BlockSpec(memory_space=pltpu.SMEM),
        ],
        out_specs=pl.BlockSpec(memory_space=pltpu.SMEM),
        out_shape=jax.ShapeDtypeStruct((1,), jnp.float32),
        scratch_shapes=[pltpu.SMEM((1,), jnp.float32)],
        interpret=interpret,
    )(hg, rg, tg, tcg, par, stats)


TAIL_C0 = NUM_ENT - ENT_BLK
TAIL_P0 = NUM_PAIR - PAIR_BLK


def _ent_pair_idx(e):
    off = e & (ENT_BLK - 1)
    std_p = (e >> 14) * PAIR_BLK + (off & (PAIR_BLK - 1))
    std_h = off >> 13
    toff = e - TAIL_C0
    tail_p = TAIL_P0 + (toff & (PAIR_BLK - 1))
    tail_h = toff >> 13
    in_std = e < TAIL_C0
    return (jnp.where(in_std, std_p, tail_p),
            jnp.where(in_std, std_h, tail_h))


def kernel(entity_table, relation_table, triplets, corrupted_triplets):
    heads = triplets[:, 0]
    rels = triplets[:, 1]
    tails = triplets[:, 2]
    ctails = corrupted_triplets[:, 2]

    ent_t = entity_table.T
    rel_pair = relation_table.reshape(NUM_REL // 2, 2 * DIM)

    pair_table, stats = _scan_repack(ent_t, rel_pair)

    hp, hb = _ent_pair_idx(heads)
    tp, tb = _ent_pair_idx(tails)
    cp, cb = _ent_pair_idx(ctails)

    hg, rg, tg, tcg = _sc_gather()(
        pair_table, rel_pair, hp, rels >> 1, tp, cp)

    par = jnp.stack([hb, rels & 1, tb, cb],
                    axis=0).astype(jnp.float32)

    out = _score(hg, rg, tg, tcg, par, stats)
    return out[0]


# speedup vs baseline: 3.4818x; 1.0409x over previous
import functools

import jax
import jax.numpy as jnp
from jax import lax
from jax.experimental import pallas as pl
from jax.experimental.pallas import tpu as pltpu
from jax.experimental.pallas import tpu_sc as plsc

NUM_ENT = 1000000
NUM_REL = 500
DIM = 64
B = 16384
MARGIN = 1.0
REG_W = 0.01

NUM_PAIR = NUM_ENT // 2

NUM_WORKERS = 32
BPW = B // NUM_WORKERS
CHUNK = BPW // 2

ENT_BLK = 32768
GRID = 31
OVERLAP = GRID * ENT_BLK - NUM_ENT
PAIR_BLK = ENT_BLK // 2

TRIP_BLK = 1024
TRIP_GRID = B // TRIP_BLK


def _scan_repack_body(ent_ref, rel_ref, pair_ref, stats_ref):
    i = pl.program_id(0)
    ones_row = jnp.ones((1, DIM), jnp.float32)

    @pl.when(i == 0)
    def _init():
        rel = rel_ref[...]
        stats_ref[0] = 0.0
        stats_ref[1] = jnp.sum(rel * rel)

    x = ent_ref[...]
    s = lax.dot_general(ones_row, x * x, (((1,), (0,)), ((), ())),
                        preferred_element_type=jnp.float32)
    contrib = jnp.minimum(s * 1e24, 1.0)
    lane = lax.broadcasted_iota(jnp.int32, (1, ENT_BLK), 1)
    fresh = jnp.logical_or(i < GRID - 1, lane >= OVERLAP)
    stats_ref[0] += jnp.sum(jnp.where(fresh, contrib, 0.0))

    eye = jnp.eye(DIM, dtype=jnp.float32)
    tdims = (((0,), (0,)), ((), ()))
    pair_ref[:, :DIM] = lax.dot_general(
        x[:, :PAIR_BLK], eye, tdims, preferred_element_type=jnp.float32)
    pair_ref[:, DIM:] = lax.dot_general(
        x[:, PAIR_BLK:], eye, tdims, preferred_element_type=jnp.float32)


def _scan_repack(ent_t, rel_pair):
    return pl.pallas_call(
        _scan_repack_body,
        grid=(GRID,),
        in_specs=[
            pl.BlockSpec((DIM, ENT_BLK), lambda i: (0, i)),
            pl.BlockSpec((NUM_REL // 2, 2 * DIM), lambda i: (0, 0)),
        ],
        out_specs=[
            pl.BlockSpec((PAIR_BLK, 2 * DIM), lambda i: (i, 0)),
            pl.BlockSpec(memory_space=pltpu.SMEM),
        ],
        out_shape=[
            jax.ShapeDtypeStruct((NUM_PAIR, 2 * DIM), jnp.float32),
            jax.ShapeDtypeStruct((2,), jnp.float32),
        ],
    )(ent_t, rel_pair)


def _sc_gather_body(pair_hbm, relp_hbm, h_hbm, r_hbm, t_hbm, tc_hbm,
                    hg_hbm, rg_hbm, tg_hbm, tcg_hbm,
                    i0, i1, i2, i3, rows0, rows1, gsem, wsem0, wsem1):
    wid = lax.axis_index("s") * 2 + lax.axis_index("c")
    base = wid * BPW
    plan = (
        (i0, h_hbm, pair_hbm, hg_hbm),
        (i1, r_hbm, relp_hbm, rg_hbm),
        (i2, t_hbm, pair_hbm, tg_hbm),
        (i3, tc_hbm, pair_hbm, tcg_hbm),
    )
    for idx_v, idx_hbm, _, _ in plan:
        pltpu.sync_copy(idx_hbm.at[pl.ds(base, BPW)], idx_v)
    rows = (rows0, rows1)
    wsems = (wsem0, wsem1)
    writes = [None, None]
    steps = [(k, half) for k in range(4) for half in range(2)]
    gathers = [None] * 8
    gathers[0] = pltpu.async_copy(
        plan[0][2].at[i0.at[pl.ds(0, CHUNK)]], rows0, gsem)
    for n, (k, half) in enumerate(steps):
        gathers[n].wait()
        if n + 1 < 8:
            k2, h2 = steps[n + 1]
            buf = (n + 1) % 2
            if writes[buf] is not None:
                writes[buf].wait()
            nxt = plan[k2]
            gathers[n + 1] = pltpu.async_copy(
                nxt[2].at[nxt[0].at[pl.ds(h2 * CHUNK, CHUNK)]],
                rows[buf], gsem)
        writes[n % 2] = pltpu.async_copy(
            rows[n % 2], plan[k][3].at[pl.ds(base + half * CHUNK, CHUNK)],
            wsems[n % 2])
    writes[0].wait()
    writes[1].wait()


@functools.cache
def _sc_gather():
    return pl.kernel(
        _sc_gather_body,
        mesh=plsc.VectorSubcoreMesh(core_axis_name="c", subcore_axis_name="s"),
        out_type=[jax.ShapeDtypeStruct((B, 2 * DIM), jnp.float32)] * 4,
        scratch_types=[
            pltpu.VMEM((BPW,), jnp.int32),
            pltpu.VMEM((BPW,), jnp.int32),
            pltpu.VMEM((BPW,), jnp.int32),
            pltpu.VMEM((BPW,), jnp.int32),
            pltpu.VMEM((CHUNK, 2 * DIM), jnp.float32),
            pltpu.VMEM((CHUNK, 2 * DIM), jnp.float32),
            pltpu.SemaphoreType.DMA,
            pltpu.SemaphoreType.DMA,
            pltpu.SemaphoreType.DMA,
        ],
    )


def _row_dots(a, b, ones_row):
    return lax.dot_general(ones_row, a * b, (((1,), (1,)), ((), ())),
                           preferred_element_type=jnp.float32)


def _score_body(hg_ref, rg_ref, tg_ref, tcg_ref, par_ref, stats_ref,
                out_ref, acc_ref):
    i = pl.program_id(0)

    @pl.when(i == 0)
    def _init():
        acc_ref[0] = 0.0

    par = jnp.transpose(par_ref[...])

    def pick(ref, col):
        blk = ref[...]
        even = blk[:, :DIM]
        odd = blk[:, DIM:]
        p = par[:, col:col + 1]
        return even + p * (odd - even)

    h = pick(hg_ref, 0)
    r = pick(rg_ref, 1)
    t = pick(tg_ref, 2)
    c = pick(tcg_ref, 3)

    ones_row = jnp.ones((1, DIM), jnp.float32)
    hh = _row_dots(h, h, ones_row)
    tt = _row_dots(t, t, ones_row)
    cc = _row_dots(c, c, ones_row)
    rr = _row_dots(r, r, ones_row)
    hr = _row_dots(h, r, ones_row)
    ht = _row_dots(h, t, ones_row)
    hc = _row_dots(h, c, ones_row)
    rt = _row_dots(r, t, ones_row)
    rc = _row_dots(r, c, ones_row)
    dh = jnp.maximum(jnp.sqrt(hh), 1e-12)
    dt = jnp.maximum(jnp.sqrt(tt), 1e-12)
    dc = jnp.maximum(jnp.sqrt(cc), 1e-12)
    nh = hh / (dh * dh)
    pos2 = (nh + tt / (dt * dt) + rr
            + 2.0 * (hr / dh - ht / (dh * dt) - rt / dt))
    neg2 = (nh + cc / (dc * dc) + rr
            + 2.0 * (hr / dh - hc / (dh * dc) - rc / dc))
    ps = -jnp.sqrt(jnp.maximum(pos2, 0.0) + 1e-12)
    ns = -jnp.sqrt(jnp.maximum(neg2, 0.0) + 1e-12)
    hinge = jnp.maximum(0.0, ns - ps + MARGIN)
    acc_ref[0] += jnp.sum(hinge)

    @pl.when(i == TRIP_GRID - 1)
    def _final():
        out_ref[0] = acc_ref[0] / B + REG_W * (stats_ref[0] + stats_ref[1])


def _score(hg, rg, tg, tcg, par, stats, interpret=False):
    gspec = pl.BlockSpec((TRIP_BLK, 2 * DIM), lambda i: (i, 0))
    return pl.pallas_call(
        _score_body,
        grid=(TRIP_GRID,),
        in_specs=[
            gspec, gspec, gspec, gspec,
            pl.BlockSpec((4, TRIP_BLK), lambda i: (0, i)),
            pl.BlockSpec(memory_space=pltpu.SMEM),
        ],
        out_specs=pl.BlockSpec(memory_space=pltpu.SMEM),
        out_shape=jax.ShapeDtypeStruct((1,), jnp.float32),
        scratch_shapes=[pltpu.SMEM((1,), jnp.float32)],
        interpret=interpret,
    )(hg, rg, tg, tcg, par, stats)


TAIL_C0 = NUM_ENT - ENT_BLK
TAIL_P0 = NUM_PAIR - PAIR_BLK


def _ent_pair_idx(e):
    off = e % ENT_BLK
    std_p = (e // ENT_BLK) * PAIR_BLK + (off % PAIR_BLK)
    std_h = off // PAIR_BLK
    toff = e - TAIL_C0
    tail_p = TAIL_P0 + (toff % PAIR_BLK)
    tail_h = toff // PAIR_BLK
    in_std = e < TAIL_C0
    return (jnp.where(in_std, std_p, tail_p),
            jnp.where(in_std, std_h, tail_h))


def kernel(entity_table, relation_table, triplets, corrupted_triplets):
    heads = triplets[:, 0]
    rels = triplets[:, 1]
    tails = triplets[:, 2]
    ctails = corrupted_triplets[:, 2]

    ent_t = entity_table.T
    rel_pair = relation_table.reshape(NUM_REL // 2, 2 * DIM)

    pair_table, stats = _scan_repack(ent_t, rel_pair)

    hp, hb = _ent_pair_idx(heads)
    tp, tb = _ent_pair_idx(tails)
    cp, cb = _ent_pair_idx(ctails)

    hg, rg, tg, tcg = _sc_gather()(
        pair_table, rel_pair, hp, rels >> 1, tp, cp)

    par = jnp.stack([hb, rels & 1, tb, cb],
                    axis=0).astype(jnp.float32)

    out = _score(hg, rg, tg, tcg, par, stats)
    return out[0]


# speedup vs baseline: 3.8143x; 1.0955x over previous
import functools

import jax
import jax.numpy as jnp
from jax import lax
from jax.experimental import pallas as pl
from jax.experimental.pallas import tpu as pltpu
from jax.experimental.pallas import tpu_sc as plsc

NUM_ENT = 1000000
NUM_REL = 500
DIM = 64
B = 16384
MARGIN = 1.0
REG_W = 0.01

NUM_PAIR = NUM_ENT // 2
NUM_QUAD = NUM_ENT // 4

NUM_WORKERS = 32
BPW = B // NUM_WORKERS
CHUNK = BPW // 2

ENT_BLK = 32768
GRID = 31
OVERLAP = GRID * ENT_BLK - NUM_ENT
QUARTER = ENT_BLK // 4

TRIP_BLK = 1024
TRIP_GRID = B // TRIP_BLK


def _scan_repack_body(ent_ref, rel_ref, pair_ref, stats_ref):
    i = pl.program_id(0)
    ones_row = jnp.ones((1, DIM), jnp.float32)

    @pl.when(i == 0)
    def _init():
        rel = rel_ref[...]
        stats_ref[0] = 0.0
        stats_ref[1] = jnp.sum(rel * rel)

    x = ent_ref[...]
    s = lax.dot_general(ones_row, x * x, (((1,), (0,)), ((), ())),
                        preferred_element_type=jnp.float32)
    contrib = jnp.minimum(s * 1e24, 1.0)
    lane = lax.broadcasted_iota(jnp.int32, (1, ENT_BLK), 1)
    fresh = jnp.logical_or(i < GRID - 1, lane >= OVERLAP)
    stats_ref[0] += jnp.sum(jnp.where(fresh, contrib, 0.0))

    eye = jnp.eye(DIM, dtype=jnp.float32)
    tdims = (((0,), (0,)), ((), ()))

    def tq(k):
        xt = lax.dot_general(x[:, k * QUARTER:(k + 1) * QUARTER], eye,
                             tdims, preferred_element_type=jnp.float32)
        bits = lax.bitcast_convert_type(
            xt.astype(jnp.bfloat16).astype(jnp.float32), jnp.uint32)
        return bits

    pair_ref[:, :DIM] = lax.bitcast_convert_type(
        tq(0) | (tq(1) >> 16), jnp.float32)
    pair_ref[:, DIM:] = lax.bitcast_convert_type(
        tq(2) | (tq(3) >> 16), jnp.float32)


def _scan_repack(ent_t, rel_pair):
    return pl.pallas_call(
        _scan_repack_body,
        grid=(GRID,),
        in_specs=[
            pl.BlockSpec((DIM, ENT_BLK), lambda i: (0, i)),
            pl.BlockSpec((NUM_REL // 2, 2 * DIM), lambda i: (0, 0)),
        ],
        out_specs=[
            pl.BlockSpec((QUARTER, 2 * DIM), lambda i: (i, 0)),
            pl.BlockSpec(memory_space=pltpu.SMEM),
        ],
        out_shape=[
            jax.ShapeDtypeStruct((NUM_QUAD, 2 * DIM), jnp.float32),
            jax.ShapeDtypeStruct((2,), jnp.float32),
        ],
    )(ent_t, rel_pair)


def _sc_gather_body(pair_hbm, relp_hbm, h_hbm, r_hbm, t_hbm, tc_hbm,
                    hg_hbm, rg_hbm, tg_hbm, tcg_hbm,
                    i0, i1, i2, i3, rows0, rows1, gsem, wsem0, wsem1):
    wid = lax.axis_index("s") * 2 + lax.axis_index("c")
    base = wid * BPW
    plan = (
        (i0, h_hbm, pair_hbm, hg_hbm),
        (i1, r_hbm, relp_hbm, rg_hbm),
        (i2, t_hbm, pair_hbm, tg_hbm),
        (i3, tc_hbm, pair_hbm, tcg_hbm),
    )
    for idx_v, idx_hbm, _, _ in plan:
        pltpu.sync_copy(idx_hbm.at[pl.ds(base, BPW)], idx_v)
    rows = (rows0, rows1)
    wsems = (wsem0, wsem1)
    writes = [None, None]
    steps = [(k, half) for k in range(4) for half in range(2)]
    gathers = [None] * 8
    gathers[0] = pltpu.async_copy(
        plan[0][2].at[i0.at[pl.ds(0, CHUNK)]], rows0, gsem)
    for n, (k, half) in enumerate(steps):
        gathers[n].wait()
        if n + 1 < 8:
            k2, h2 = steps[n + 1]
            buf = (n + 1) % 2
            if writes[buf] is not None:
                writes[buf].wait()
            nxt = plan[k2]
            gathers[n + 1] = pltpu.async_copy(
                nxt[2].at[nxt[0].at[pl.ds(h2 * CHUNK, CHUNK)]],
                rows[buf], gsem)
        writes[n % 2] = pltpu.async_copy(
            rows[n % 2], plan[k][3].at[pl.ds(base + half * CHUNK, CHUNK)],
            wsems[n % 2])
    writes[0].wait()
    writes[1].wait()


@functools.cache
def _sc_gather():
    return pl.kernel(
        _sc_gather_body,
        mesh=plsc.VectorSubcoreMesh(core_axis_name="c", subcore_axis_name="s"),
        out_type=[jax.ShapeDtypeStruct((B, 2 * DIM), jnp.float32)] * 4,
        scratch_types=[
            pltpu.VMEM((BPW,), jnp.int32),
            pltpu.VMEM((BPW,), jnp.int32),
            pltpu.VMEM((BPW,), jnp.int32),
            pltpu.VMEM((BPW,), jnp.int32),
            pltpu.VMEM((CHUNK, 2 * DIM), jnp.float32),
            pltpu.VMEM((CHUNK, 2 * DIM), jnp.float32),
            pltpu.SemaphoreType.DMA,
            pltpu.SemaphoreType.DMA,
            pltpu.SemaphoreType.DMA,
        ],
    )


def _row_dots(a, b, ones_row):
    return lax.dot_general(ones_row, a * b, (((1,), (1,)), ((), ())),
                           preferred_element_type=jnp.float32)


def _score_body(hg_ref, rg_ref, tg_ref, tcg_ref, par_ref, stats_ref,
                out_ref, acc_ref):
    i = pl.program_id(0)

    @pl.when(i == 0)
    def _init():
        acc_ref[0] = 0.0

    par = jnp.transpose(par_ref[...])

    def pick_quad(ref, half_col, word_col):
        blk = ref[...]
        p_half = par[:, half_col:half_col + 1] > 0.5
        hw = jnp.where(p_half, blk[:, DIM:], blk[:, :DIM])
        u = lax.bitcast_convert_type(hw, jnp.uint32)
        p_word = par[:, word_col:word_col + 1] > 0.5
        lo = lax.bitcast_convert_type(u << 16, jnp.float32)
        hi = lax.bitcast_convert_type(u & jnp.uint32(0xFFFF0000),
                                      jnp.float32)
        return jnp.where(p_word, lo, hi)

    def pick_rel(ref, col):
        blk = ref[...]
        even = blk[:, :DIM]
        odd = blk[:, DIM:]
        p = par[:, col:col + 1]
        return even + p * (odd - even)

    h = pick_quad(hg_ref, 0, 1)
    r = pick_rel(rg_ref, 2)
    t = pick_quad(tg_ref, 4, 5)
    c = pick_quad(tcg_ref, 6, 7)

    ones_row = jnp.ones((1, DIM), jnp.float32)
    hh = _row_dots(h, h, ones_row)
    tt = _row_dots(t, t, ones_row)
    cc = _row_dots(c, c, ones_row)
    rr = _row_dots(r, r, ones_row)
    hr = _row_dots(h, r, ones_row)
    ht = _row_dots(h, t, ones_row)
    hc = _row_dots(h, c, ones_row)
    rt = _row_dots(r, t, ones_row)
    rc = _row_dots(r, c, ones_row)
    dh = jnp.maximum(jnp.sqrt(hh), 1e-12)
    dt = jnp.maximum(jnp.sqrt(tt), 1e-12)
    dc = jnp.maximum(jnp.sqrt(cc), 1e-12)
    nh = hh / (dh * dh)
    pos2 = (nh + tt / (dt * dt) + rr
            + 2.0 * (hr / dh - ht / (dh * dt) - rt / dt))
    neg2 = (nh + cc / (dc * dc) + rr
            + 2.0 * (hr / dh - hc / (dh * dc) - rc / dc))
    ps = -jnp.sqrt(jnp.maximum(pos2, 0.0) + 1e-12)
    ns = -jnp.sqrt(jnp.maximum(neg2, 0.0) + 1e-12)
    hinge = jnp.maximum(0.0, ns - ps + MARGIN)
    acc_ref[0] += jnp.sum(hinge)

    @pl.when(i == TRIP_GRID - 1)
    def _final():
        out_ref[0] = acc_ref[0] / B + REG_W * (stats_ref[0] + stats_ref[1])


def _score(hg, rg, tg, tcg, par, stats, interpret=False):
    gspec = pl.BlockSpec((TRIP_BLK, 2 * DIM), lambda i: (i, 0))
    return pl.pallas_call(
        _score_body,
        grid=(TRIP_GRID,),
        in_specs=[
            gspec, gspec, gspec, gspec,
            pl.BlockSpec((8, TRIP_BLK), lambda i: (0, i)),
            pl.BlockSpec(memory_space=pltpu.SMEM),
        ],
        out_specs=pl.BlockSpec(memory_space=pltpu.SMEM),
        out_shape=jax.ShapeDtypeStruct((1,), jnp.float32),
        scratch_shapes=[pltpu.SMEM((1,), jnp.float32)],
        interpret=interpret,
    )(hg, rg, tg, tcg, par, stats)


TAIL_C0 = NUM_ENT - ENT_BLK
TAIL_Q0 = NUM_QUAD - QUARTER


def _ent_quad_idx(e):
    off = e % ENT_BLK
    std_q = (e // ENT_BLK) * QUARTER + (off % QUARTER)
    std_k = off // QUARTER
    toff = e - TAIL_C0
    tail_q = TAIL_Q0 + (toff % QUARTER)
    tail_k = toff // QUARTER
    in_std = e < TAIL_C0
    q = jnp.where(in_std, std_q, tail_q)
    k = jnp.where(in_std, std_k, tail_k)
    return q, k >> 1, k & 1


def kernel(entity_table, relation_table, triplets, corrupted_triplets):
    heads = triplets[:, 0]
    rels = triplets[:, 1]
    tails = triplets[:, 2]
    ctails = corrupted_triplets[:, 2]

    ent_t = entity_table.T
    rel_pair = relation_table.reshape(NUM_REL // 2, 2 * DIM)

    quad_table, stats = _scan_repack(ent_t, rel_pair)

    hq, hh, hw = _ent_quad_idx(heads)
    tq_, th, tw = _ent_quad_idx(tails)
    cq, ch, cw = _ent_quad_idx(ctails)

    hg, rg, tg, tcg = _sc_gather()(
        quad_table, rel_pair, hq, rels >> 1, tq_, cq)

    zero = jnp.zeros((B,), jnp.int32)
    par = jnp.stack([hh, hw, rels & 1, zero, th, tw, ch, cw],
                    axis=0).astype(jnp.float32)

    out = _score(hg, rg, tg, tcg, par, stats)
    return out[0]


# speedup vs baseline: 3.8565x; 1.0111x over previous
import functools

import jax
import jax.numpy as jnp
from jax import lax
from jax.experimental import pallas as pl
from jax.experimental.pallas import tpu as pltpu
from jax.experimental.pallas import tpu_sc as plsc

NUM_ENT = 1000000
NUM_REL = 500
DIM = 64
B = 16384
MARGIN = 1.0
REG_W = 0.01

NUM_PAIR = NUM_ENT // 2
NUM_QUAD = NUM_ENT // 4

NUM_WORKERS = 32
BPW = B // NUM_WORKERS
CHUNK = BPW // 2

ENT_BLK = 32768
GRID = 31
OVERLAP = GRID * ENT_BLK - NUM_ENT
QUARTER = ENT_BLK // 4

TRIP_BLK = 2048
TRIP_GRID = B // TRIP_BLK


def _scan_repack_body(ent_ref, rel_ref, pair_ref, stats_ref):
    i = pl.program_id(0)
    ones_row = jnp.ones((1, DIM), jnp.float32)

    @pl.when(i == 0)
    def _init():
        rel = rel_ref[...]
        stats_ref[0] = 0.0
        stats_ref[1] = jnp.sum(rel * rel)

    x = ent_ref[...]
    s = lax.dot_general(ones_row, x * x, (((1,), (0,)), ((), ())),
                        preferred_element_type=jnp.float32)
    contrib = jnp.minimum(s * 1e24, 1.0)
    lane = lax.broadcasted_iota(jnp.int32, (1, ENT_BLK), 1)
    fresh = jnp.logical_or(i < GRID - 1, lane >= OVERLAP)
    stats_ref[0] += jnp.sum(jnp.where(fresh, contrib, 0.0))

    eye = jnp.eye(DIM, dtype=jnp.float32)
    tdims = (((0,), (0,)), ((), ()))

    def tq(k):
        xt = lax.dot_general(x[:, k * QUARTER:(k + 1) * QUARTER], eye,
                             tdims, preferred_element_type=jnp.float32)
        bits = lax.bitcast_convert_type(
            xt.astype(jnp.bfloat16).astype(jnp.float32), jnp.uint32)
        return bits

    pair_ref[:, :DIM] = lax.bitcast_convert_type(
        tq(0) | (tq(1) >> 16), jnp.float32)
    pair_ref[:, DIM:] = lax.bitcast_convert_type(
        tq(2) | (tq(3) >> 16), jnp.float32)


def _scan_repack(ent_t, rel_pair):
    return pl.pallas_call(
        _scan_repack_body,
        grid=(GRID,),
        in_specs=[
            pl.BlockSpec((DIM, ENT_BLK), lambda i: (0, i)),
            pl.BlockSpec((NUM_REL // 2, 2 * DIM), lambda i: (0, 0)),
        ],
        out_specs=[
            pl.BlockSpec((QUARTER, 2 * DIM), lambda i: (i, 0)),
            pl.BlockSpec(memory_space=pltpu.SMEM),
        ],
        out_shape=[
            jax.ShapeDtypeStruct((NUM_QUAD, 2 * DIM), jnp.float32),
            jax.ShapeDtypeStruct((2,), jnp.float32),
        ],
    )(ent_t, rel_pair)


def _sc_gather_body(pair_hbm, relp_hbm, h_hbm, r_hbm, t_hbm, tc_hbm,
                    hg_hbm, rg_hbm, tg_hbm, tcg_hbm,
                    i0, i1, i2, i3, rows0, rows1, gsem, wsem0, wsem1):
    wid = lax.axis_index("s") * 2 + lax.axis_index("c")
    base = wid * BPW
    plan = (
        (i0, h_hbm, pair_hbm, hg_hbm),
        (i1, r_hbm, relp_hbm, rg_hbm),
        (i2, t_hbm, pair_hbm, tg_hbm),
        (i3, tc_hbm, pair_hbm, tcg_hbm),
    )
    for idx_v, idx_hbm, _, _ in plan:
        pltpu.sync_copy(idx_hbm.at[pl.ds(base, BPW)], idx_v)
    rows = (rows0, rows1)
    wsems = (wsem0, wsem1)
    writes = [None, None]
    steps = [(k, half) for k in range(4) for half in range(2)]
    gathers = [None] * 8
    gathers[0] = pltpu.async_copy(
        plan[0][2].at[i0.at[pl.ds(0, CHUNK)]], rows0, gsem)
    for n, (k, half) in enumerate(steps):
        gathers[n].wait()
        if n + 1 < 8:
            k2, h2 = steps[n + 1]
            buf = (n + 1) % 2
            if writes[buf] is not None:
                writes[buf].wait()
            nxt = plan[k2]
            gathers[n + 1] = pltpu.async_copy(
                nxt[2].at[nxt[0].at[pl.ds(h2 * CHUNK, CHUNK)]],
                rows[buf], gsem)
        writes[n % 2] = pltpu.async_copy(
            rows[n % 2], plan[k][3].at[pl.ds(base + half * CHUNK, CHUNK)],
            wsems[n % 2])
    writes[0].wait()
    writes[1].wait()


@functools.cache
def _sc_gather():
    return pl.kernel(
        _sc_gather_body,
        mesh=plsc.VectorSubcoreMesh(core_axis_name="c", subcore_axis_name="s"),
        out_type=[jax.ShapeDtypeStruct((B, 2 * DIM), jnp.float32)] * 4,
        scratch_types=[
            pltpu.VMEM((BPW,), jnp.int32),
            pltpu.VMEM((BPW,), jnp.int32),
            pltpu.VMEM((BPW,), jnp.int32),
            pltpu.VMEM((BPW,), jnp.int32),
            pltpu.VMEM((CHUNK, 2 * DIM), jnp.float32),
            pltpu.VMEM((CHUNK, 2 * DIM), jnp.float32),
            pltpu.SemaphoreType.DMA,
            pltpu.SemaphoreType.DMA,
            pltpu.SemaphoreType.DMA,
        ],
    )


def _row_dots(a, b, ones_row):
    return lax.dot_general(ones_row, a * b, (((1,), (1,)), ((), ())),
                           preferred_element_type=jnp.float32)


def _score_body(hg_ref, rg_ref, tg_ref, tcg_ref, par_ref, stats_ref,
                out_ref, acc_ref):
    i = pl.program_id(0)

    @pl.when(i == 0)
    def _init():
        acc_ref[0] = 0.0

    par = jnp.transpose(par_ref[...])

    def pick_quad(ref, half_col, word_col):
        blk = ref[...]
        p_half = par[:, half_col:half_col + 1] > 0.5
        hw = jnp.where(p_half, blk[:, DIM:], blk[:, :DIM])
        u = lax.bitcast_convert_type(hw, jnp.uint32)
        p_word = par[:, word_col:word_col + 1] > 0.5
        lo = lax.bitcast_convert_type(u << 16, jnp.float32)
        hi = lax.bitcast_convert_type(u & jnp.uint32(0xFFFF0000),
                                      jnp.float32)
        return jnp.where(p_word, lo, hi)

    def pick_rel(ref, col):
        blk = ref[...]
        even = blk[:, :DIM]
        odd = blk[:, DIM:]
        p = par[:, col:col + 1]
        return even + p * (odd - even)

    h = pick_quad(hg_ref, 0, 1)
    r = pick_rel(rg_ref, 2)
    t = pick_quad(tg_ref, 4, 5)
    c = pick_quad(tcg_ref, 6, 7)

    ones_row = jnp.ones((1, DIM), jnp.float32)
    hh = _row_dots(h, h, ones_row)
    tt = _row_dots(t, t, ones_row)
    cc = _row_dots(c, c, ones_row)
    rr = _row_dots(r, r, ones_row)
    hr = _row_dots(h, r, ones_row)
    ht = _row_dots(h, t, ones_row)
    hc = _row_dots(h, c, ones_row)
    rt = _row_dots(r, t, ones_row)
    rc = _row_dots(r, c, ones_row)
    dh = jnp.maximum(jnp.sqrt(hh), 1e-12)
    dt = jnp.maximum(jnp.sqrt(tt), 1e-12)
    dc = jnp.maximum(jnp.sqrt(cc), 1e-12)
    nh = hh / (dh * dh)
    pos2 = (nh + tt / (dt * dt) + rr
            + 2.0 * (hr / dh - ht / (dh * dt) - rt / dt))
    neg2 = (nh + cc / (dc * dc) + rr
            + 2.0 * (hr / dh - hc / (dh * dc) - rc / dc))
    ps = -jnp.sqrt(jnp.maximum(pos2, 0.0) + 1e-12)
    ns = -jnp.sqrt(jnp.maximum(neg2, 0.0) + 1e-12)
    hinge = jnp.maximum(0.0, ns - ps + MARGIN)
    acc_ref[0] += jnp.sum(hinge)

    @pl.when(i == TRIP_GRID - 1)
    def _final():
        out_ref[0] = acc_ref[0] / B + REG_W * (stats_ref[0] + stats_ref[1])


def _score(hg, rg, tg, tcg, par, stats, interpret=False):
    gspec = pl.BlockSpec((TRIP_BLK, 2 * DIM), lambda i: (i, 0))
    return pl.pallas_call(
        _score_body,
        grid=(TRIP_GRID,),
        in_specs=[
            gspec, gspec, gspec, gspec,
            pl.BlockSpec((8, TRIP_BLK), lambda i: (0, i)),
            pl.BlockSpec(memory_space=pltpu.SMEM),
        ],
        out_specs=pl.BlockSpec(memory_space=pltpu.SMEM),
        out_shape=jax.ShapeDtypeStruct((1,), jnp.float32),
        scratch_shapes=[pltpu.SMEM((1,), jnp.float32)],
        interpret=interpret,
    )(hg, rg, tg, tcg, par, stats)


TAIL_C0 = NUM_ENT - ENT_BLK
TAIL_Q0 = NUM_QUAD - QUARTER


def _ent_quad_idx(e):
    off = e % ENT_BLK
    std_q = (e // ENT_BLK) * QUARTER + (off % QUARTER)
    std_k = off // QUARTER
    toff = e - TAIL_C0
    tail_q = TAIL_Q0 + (toff % QUARTER)
    tail_k = toff // QUARTER
    in_std = e < TAIL_C0
    q = jnp.where(in_std, std_q, tail_q)
    k = jnp.where(in_std, std_k, tail_k)
    return q, k >> 1, k & 1


def kernel(entity_table, relation_table, triplets, corrupted_triplets):
    heads = triplets[:, 0]
    rels = triplets[:, 1]
    tails = triplets[:, 2]
    ctails = corrupted_triplets[:, 2]

    ent_t = entity_table.T
    rel_pair = relation_table.reshape(NUM_REL // 2, 2 * DIM)

    quad_table, stats = _scan_repack(ent_t, rel_pair)

    hq, hh, hw = _ent_quad_idx(heads)
    tq_, th, tw = _ent_quad_idx(tails)
    cq, ch, cw = _ent_quad_idx(ctails)

    hg, rg, tg, tcg = _sc_gather()(
        quad_table, rel_pair, hq, rels >> 1, tq_, cq)

    zero = jnp.zeros((B,), jnp.int32)
    par = jnp.stack([hh, hw, rels & 1, zero, th, tw, ch, cw],
                    axis=0).astype(jnp.float32)

    out = _score(hg, rg, tg, tcg, par, stats)
    return out[0]


# speedup vs baseline: 4.5112x; 1.1698x over previous
import functools

import jax
import jax.numpy as jnp
from jax import lax
from jax.experimental import pallas as pl
from jax.experimental.pallas import tpu as pltpu
from jax.experimental.pallas import tpu_sc as plsc

NUM_ENT = 1000000
NUM_REL = 500
DIM = 64
B = 16384
MARGIN = 1.0
REG_W = 0.01

NUM_PAIR = NUM_ENT // 2
NUM_QUAD = NUM_ENT // 4

NUM_WORKERS = 32
BPW = B // NUM_WORKERS
CHUNK = BPW // 2

ENT_BLK = 32768
GRID = 31
OVERLAP = GRID * ENT_BLK - NUM_ENT
QUARTER = ENT_BLK // 4

TRIP_BLK = 2048
TRIP_GRID = B // TRIP_BLK


def _scan_repack_body(ent_ref, rel_ref, pair_ref, stats_ref):
    i = pl.program_id(0)
    ones_row = jnp.ones((1, DIM), jnp.float32)

    @pl.when(i == 0)
    def _init():
        rel = rel_ref[...]
        stats_ref[0] = 0.0
        stats_ref[1] = jnp.sum(rel * rel)

    x = ent_ref[...]
    s = lax.dot_general(ones_row, x * x, (((1,), (0,)), ((), ())),
                        preferred_element_type=jnp.float32)
    contrib = jnp.minimum(s * 1e24, 1.0)
    lane = lax.broadcasted_iota(jnp.int32, (1, ENT_BLK), 1)
    fresh = jnp.logical_or(i < GRID - 1, lane >= OVERLAP)
    stats_ref[0] += jnp.sum(jnp.where(fresh, contrib, 0.0))

    u = lax.bitcast_convert_type(x, jnp.uint32)
    hi_mask = jnp.uint32(0xFFFF0000)

    def packed_t(k0, k1):
        w = ((u[:, k0 * QUARTER:(k0 + 1) * QUARTER] & hi_mask)
             | (u[:, k1 * QUARTER:(k1 + 1) * QUARTER] >> 16))
        return lax.bitcast_convert_type(jnp.transpose(w), jnp.float32)

    pair_ref[:, :DIM] = packed_t(0, 1)
    pair_ref[:, DIM:] = packed_t(2, 3)


def _scan_repack(ent_t, rel_pair):
    return pl.pallas_call(
        _scan_repack_body,
        grid=(GRID,),
        in_specs=[
            pl.BlockSpec((DIM, ENT_BLK), lambda i: (0, i)),
            pl.BlockSpec((NUM_REL // 2, 2 * DIM), lambda i: (0, 0)),
        ],
        out_specs=[
            pl.BlockSpec((QUARTER, 2 * DIM), lambda i: (i, 0)),
            pl.BlockSpec(memory_space=pltpu.SMEM),
        ],
        out_shape=[
            jax.ShapeDtypeStruct((NUM_QUAD, 2 * DIM), jnp.float32),
            jax.ShapeDtypeStruct((2,), jnp.float32),
        ],
    )(ent_t, rel_pair)


def _sc_gather_body(pair_hbm, relp_hbm, h_hbm, r_hbm, t_hbm, tc_hbm,
                    hg_hbm, rg_hbm, tg_hbm, tcg_hbm,
                    i0, i1, i2, i3, rows0, rows1, gsem, wsem0, wsem1):
    wid = lax.axis_index("s") * 2 + lax.axis_index("c")
    base = wid * BPW
    plan = (
        (i0, h_hbm, pair_hbm, hg_hbm),
        (i1, r_hbm, relp_hbm, rg_hbm),
        (i2, t_hbm, pair_hbm, tg_hbm),
        (i3, tc_hbm, pair_hbm, tcg_hbm),
    )
    for idx_v, idx_hbm, _, _ in plan:
        pltpu.sync_copy(idx_hbm.at[pl.ds(base, BPW)], idx_v)
    rows = (rows0, rows1)
    wsems = (wsem0, wsem1)
    writes = [None, None]
    steps = [(k, half) for k in range(4) for half in range(2)]
    gathers = [None] * 8
    gathers[0] = pltpu.async_copy(
        plan[0][2].at[i0.at[pl.ds(0, CHUNK)]], rows0, gsem)
    for n, (k, half) in enumerate(steps):
        gathers[n].wait()
        if n + 1 < 8:
            k2, h2 = steps[n + 1]
            buf = (n + 1) % 2
            if writes[buf] is not None:
                writes[buf].wait()
            nxt = plan[k2]
            gathers[n + 1] = pltpu.async_copy(
                nxt[2].at[nxt[0].at[pl.ds(h2 * CHUNK, CHUNK)]],
                rows[buf], gsem)
        writes[n % 2] = pltpu.async_copy(
            rows[n % 2], plan[k][3].at[pl.ds(base + half * CHUNK, CHUNK)],
            wsems[n % 2])
    writes[0].wait()
    writes[1].wait()


@functools.cache
def _sc_gather():
    return pl.kernel(
        _sc_gather_body,
        mesh=plsc.VectorSubcoreMesh(core_axis_name="c", subcore_axis_name="s"),
        out_type=[jax.ShapeDtypeStruct((B, 2 * DIM), jnp.float32)] * 4,
        scratch_types=[
            pltpu.VMEM((BPW,), jnp.int32),
            pltpu.VMEM((BPW,), jnp.int32),
            pltpu.VMEM((BPW,), jnp.int32),
            pltpu.VMEM((BPW,), jnp.int32),
            pltpu.VMEM((CHUNK, 2 * DIM), jnp.float32),
            pltpu.VMEM((CHUNK, 2 * DIM), jnp.float32),
            pltpu.SemaphoreType.DMA,
            pltpu.SemaphoreType.DMA,
            pltpu.SemaphoreType.DMA,
        ],
    )


def _row_dots(a, b, ones_row):
    return lax.dot_general(ones_row, a * b, (((1,), (1,)), ((), ())),
                           preferred_element_type=jnp.float32)


def _score_body(hg_ref, rg_ref, tg_ref, tcg_ref, par_ref, stats_ref,
                out_ref, acc_ref):
    i = pl.program_id(0)

    @pl.when(i == 0)
    def _init():
        acc_ref[0] = 0.0

    par = jnp.transpose(par_ref[...])

    def pick_quad(ref, half_col, word_col):
        blk = ref[...]
        p_half = par[:, half_col:half_col + 1] > 0.5
        hw = jnp.where(p_half, blk[:, DIM:], blk[:, :DIM])
        u = lax.bitcast_convert_type(hw, jnp.uint32)
        p_word = par[:, word_col:word_col + 1] > 0.5
        lo = lax.bitcast_convert_type(u << 16, jnp.float32)
        hi = lax.bitcast_convert_type(u & jnp.uint32(0xFFFF0000),
                                      jnp.float32)
        return jnp.where(p_word, lo, hi)

    def pick_rel(ref, col):
        blk = ref[...]
        even = blk[:, :DIM]
        odd = blk[:, DIM:]
        p = par[:, col:col + 1]
        return even + p * (odd - even)

    h = pick_quad(hg_ref, 0, 1)
    r = pick_rel(rg_ref, 2)
    t = pick_quad(tg_ref, 4, 5)
    c = pick_quad(tcg_ref, 6, 7)

    ones_row = jnp.ones((1, DIM), jnp.float32)
    hh = _row_dots(h, h, ones_row)
    tt = _row_dots(t, t, ones_row)
    cc = _row_dots(c, c, ones_row)
    rr = _row_dots(r, r, ones_row)
    hr = _row_dots(h, r, ones_row)
    ht = _row_dots(h, t, ones_row)
    hc = _row_dots(h, c, ones_row)
    rt = _row_dots(r, t, ones_row)
    rc = _row_dots(r, c, ones_row)
    dh = jnp.maximum(jnp.sqrt(hh), 1e-12)
    dt = jnp.maximum(jnp.sqrt(tt), 1e-12)
    dc = jnp.maximum(jnp.sqrt(cc), 1e-12)
    nh = hh / (dh * dh)
    pos2 = (nh + tt / (dt * dt) + rr
            + 2.0 * (hr / dh - ht / (dh * dt) - rt / dt))
    neg2 = (nh + cc / (dc * dc) + rr
            + 2.0 * (hr / dh - hc / (dh * dc) - rc / dc))
    ps = -jnp.sqrt(jnp.maximum(pos2, 0.0) + 1e-12)
    ns = -jnp.sqrt(jnp.maximum(neg2, 0.0) + 1e-12)
    hinge = jnp.maximum(0.0, ns - ps + MARGIN)
    acc_ref[0] += jnp.sum(hinge)

    @pl.when(i == TRIP_GRID - 1)
    def _final():
        out_ref[0] = acc_ref[0] / B + REG_W * (stats_ref[0] + stats_ref[1])


def _score(hg, rg, tg, tcg, par, stats, interpret=False):
    gspec = pl.BlockSpec((TRIP_BLK, 2 * DIM), lambda i: (i, 0))
    return pl.pallas_call(
        _score_body,
        grid=(TRIP_GRID,),
        in_specs=[
            gspec, gspec, gspec, gspec,
            pl.BlockSpec((8, TRIP_BLK), lambda i: (0, i)),
            pl.BlockSpec(memory_space=pltpu.SMEM),
        ],
        out_specs=pl.BlockSpec(memory_space=pltpu.SMEM),
        out_shape=jax.ShapeDtypeStruct((1,), jnp.float32),
        scratch_shapes=[pltpu.SMEM((1,), jnp.float32)],
        interpret=interpret,
    )(hg, rg, tg, tcg, par, stats)


TAIL_C0 = NUM_ENT - ENT_BLK
TAIL_Q0 = NUM_QUAD - QUARTER


def _ent_quad_idx(e):
    off = e % ENT_BLK
    std_q = (e // ENT_BLK) * QUARTER + (off % QUARTER)
    std_k = off // QUARTER
    toff = e - TAIL_C0
    tail_q = TAIL_Q0 + (toff % QUARTER)
    tail_k = toff // QUARTER
    in_std = e < TAIL_C0
    q = jnp.where(in_std, std_q, tail_q)
    k = jnp.where(in_std, std_k, tail_k)
    return q, k >> 1, k & 1


def kernel(entity_table, relation_table, triplets, corrupted_triplets):
    heads = triplets[:, 0]
    rels = triplets[:, 1]
    tails = triplets[:, 2]
    ctails = corrupted_triplets[:, 2]

    ent_t = entity_table.T
    rel_pair = relation_table.reshape(NUM_REL // 2, 2 * DIM)

    quad_table, stats = _scan_repack(ent_t, rel_pair)

    hq, hh, hw = _ent_quad_idx(heads)
    tq_, th, tw = _ent_quad_idx(tails)
    cq, ch, cw = _ent_quad_idx(ctails)

    hg, rg, tg, tcg = _sc_gather()(
        quad_table, rel_pair, hq, rels >> 1, tq_, cq)

    zero = jnp.zeros((B,), jnp.int32)
    par = jnp.stack([hh, hw, rels & 1, zero, th, tw, ch, cw],
                    axis=0).astype(jnp.float32)

    out = _score(hg, rg, tg, tcg, par, stats)
    return out[0]


# speedup vs baseline: 4.6665x; 1.0344x over previous
import functools

import jax
import jax.numpy as jnp
from jax import lax
from jax.experimental import pallas as pl
from jax.experimental.pallas import tpu as pltpu
from jax.experimental.pallas import tpu_sc as plsc

NUM_ENT = 1000000
NUM_REL = 500
DIM = 64
B = 16384
MARGIN = 1.0
REG_W = 0.01

NUM_PAIR = NUM_ENT // 2
NUM_QUAD = NUM_ENT // 4

NUM_WORKERS = 32
BPW = B // NUM_WORKERS
CHUNK = BPW // 2
HALF_B = B // 2
BPW_E = HALF_B // NUM_WORKERS
CHUNK_E = BPW_E // 2

ENT_BLK = 32768
GRID = 31
OVERLAP = GRID * ENT_BLK - NUM_ENT
QUARTER = ENT_BLK // 4

TRIP_BLK = 2048
TRIP_GRID = HALF_B // TRIP_BLK


def _scan_repack_body(ent_ref, rel_ref, pair_ref, stats_ref):
    i = pl.program_id(0)
    ones_row = jnp.ones((1, DIM), jnp.float32)

    @pl.when(i == 0)
    def _init():
        rel = rel_ref[...]
        stats_ref[0] = 0.0
        stats_ref[1] = jnp.sum(rel * rel)

    x = ent_ref[...]
    s = lax.dot_general(ones_row, x * x, (((1,), (0,)), ((), ())),
                        preferred_element_type=jnp.float32)
    contrib = jnp.minimum(s * 1e24, 1.0)
    lane = lax.broadcasted_iota(jnp.int32, (1, ENT_BLK), 1)
    fresh = jnp.logical_or(i < GRID - 1, lane >= OVERLAP)
    stats_ref[0] += jnp.sum(jnp.where(fresh, contrib, 0.0))

    u = lax.bitcast_convert_type(x, jnp.uint32)
    hi_mask = jnp.uint32(0xFFFF0000)

    def packed_t(k0, k1):
        w = ((u[:, k0 * QUARTER:(k0 + 1) * QUARTER] & hi_mask)
             | (u[:, k1 * QUARTER:(k1 + 1) * QUARTER] >> 16))
        return lax.bitcast_convert_type(jnp.transpose(w), jnp.float32)

    pair_ref[:, :DIM] = packed_t(0, 1)
    pair_ref[:, DIM:] = packed_t(2, 3)


def _scan_repack(ent_t, rel_pair):
    return pl.pallas_call(
        _scan_repack_body,
        grid=(GRID,),
        in_specs=[
            pl.BlockSpec((DIM, ENT_BLK), lambda i: (0, i)),
            pl.BlockSpec((NUM_REL // 2, 2 * DIM), lambda i: (0, 0)),
        ],
        out_specs=[
            pl.BlockSpec((QUARTER, 2 * DIM), lambda i: (i, 0)),
            pl.BlockSpec(memory_space=pltpu.SMEM),
        ],
        out_shape=[
            jax.ShapeDtypeStruct((NUM_QUAD, 2 * DIM), jnp.float32),
            jax.ShapeDtypeStruct((2,), jnp.float32),
        ],
    )(ent_t, rel_pair)


def _gather_pipeline(plan, bpw, chunk, rows, wsems, gsem):
    nsub = 2 * len(plan)
    writes = [None, None]
    steps = [(k, half) for k in range(len(plan)) for half in range(2)]
    gathers = [None] * nsub
    first = plan[0]
    gathers[0] = pltpu.async_copy(
        first[1].at[first[0].at[pl.ds(0, chunk)]], rows[0], gsem)
    for n, (k, half) in enumerate(steps):
        gathers[n].wait()
        if n + 1 < nsub:
            k2, h2 = steps[n + 1]
            buf = (n + 1) % 2
            if writes[buf] is not None:
                writes[buf].wait()
            nxt = plan[k2]
            gathers[n + 1] = pltpu.async_copy(
                nxt[1].at[nxt[0].at[pl.ds(h2 * chunk, chunk)]],
                rows[buf], gsem)
        base, out = plan[k][2], plan[k][3]
        writes[n % 2] = pltpu.async_copy(
            rows[n % 2], out.at[pl.ds(base + half * chunk, chunk)],
            wsems[n % 2])
    writes[0].wait()
    writes[1].wait()


def _sc_rel_body(relp_hbm, r_hbm, rg_hbm, i0, rows0, rows1,
                 gsem, wsem0, wsem1):
    wid = lax.axis_index("s") * 2 + lax.axis_index("c")
    base = wid * BPW
    pltpu.sync_copy(r_hbm.at[pl.ds(base, BPW)], i0)
    _gather_pipeline(((i0, relp_hbm, base, rg_hbm),), BPW, CHUNK,
                     (rows0, rows1), (wsem0, wsem1), gsem)


def _sc_ent_body(pair_hbm, h_hbm, t_hbm, tc_hbm,
                 hg_hbm, tg_hbm, tcg_hbm,
                 i0, i1, i2, rows0, rows1, gsem, wsem0, wsem1):
    wid = lax.axis_index("s") * 2 + lax.axis_index("c")
    base = wid * BPW_E
    for idx_v, idx_hbm in ((i0, h_hbm), (i1, t_hbm), (i2, tc_hbm)):
        pltpu.sync_copy(idx_hbm.at[pl.ds(base, BPW_E)], idx_v)
    plan = (
        (i0, pair_hbm, base, hg_hbm),
        (i1, pair_hbm, base, tg_hbm),
        (i2, pair_hbm, base, tcg_hbm),
    )
    _gather_pipeline(plan, BPW_E, CHUNK_E, (rows0, rows1),
                     (wsem0, wsem1), gsem)


_SC_MESH = dict(core_axis_name="c", subcore_axis_name="s")


@functools.cache
def _sc_rel():
    return pl.kernel(
        _sc_rel_body,
        mesh=plsc.VectorSubcoreMesh(**_SC_MESH),
        out_type=jax.ShapeDtypeStruct((B, 2 * DIM), jnp.float32),
        scratch_types=[
            pltpu.VMEM((BPW,), jnp.int32),
            pltpu.VMEM((CHUNK, 2 * DIM), jnp.float32),
            pltpu.VMEM((CHUNK, 2 * DIM), jnp.float32),
            pltpu.SemaphoreType.DMA,
            pltpu.SemaphoreType.DMA,
            pltpu.SemaphoreType.DMA,
        ],
    )


@functools.cache
def _sc_ent():
    return pl.kernel(
        _sc_ent_body,
        mesh=plsc.VectorSubcoreMesh(**_SC_MESH),
        out_type=[jax.ShapeDtypeStruct((HALF_B, 2 * DIM), jnp.float32)] * 3,
        scratch_types=[
            pltpu.VMEM((BPW_E,), jnp.int32),
            pltpu.VMEM((BPW_E,), jnp.int32),
            pltpu.VMEM((BPW_E,), jnp.int32),
            pltpu.VMEM((CHUNK_E, 2 * DIM), jnp.float32),
            pltpu.VMEM((CHUNK_E, 2 * DIM), jnp.float32),
            pltpu.SemaphoreType.DMA,
            pltpu.SemaphoreType.DMA,
            pltpu.SemaphoreType.DMA,
        ],
    )


def _row_dots(a, b, ones_row):
    return lax.dot_general(ones_row, a * b, (((1,), (1,)), ((), ())),
                           preferred_element_type=jnp.float32)


def _score_body(hg_ref, rg_ref, tg_ref, tcg_ref, par_ref, stats_ref,
                prev_ref, out_ref, acc_ref, *, final):
    i = pl.program_id(0)

    @pl.when(i == 0)
    def _init():
        acc_ref[0] = prev_ref[0]

    par = jnp.transpose(par_ref[...])

    def pick_quad(ref, half_col, word_col):
        blk = ref[...]
        p_half = par[:, half_col:half_col + 1] > 0.5
        hw = jnp.where(p_half, blk[:, DIM:], blk[:, :DIM])
        u = lax.bitcast_convert_type(hw, jnp.uint32)
        p_word = par[:, word_col:word_col + 1] > 0.5
        lo = lax.bitcast_convert_type(u << 16, jnp.float32)
        hi = lax.bitcast_convert_type(u & jnp.uint32(0xFFFF0000),
                                      jnp.float32)
        return jnp.where(p_word, lo, hi)

    def pick_rel(ref, col):
        blk = ref[...]
        even = blk[:, :DIM]
        odd = blk[:, DIM:]
        p = par[:, col:col + 1]
        return even + p * (odd - even)

    h = pick_quad(hg_ref, 0, 1)
    r = pick_rel(rg_ref, 2)
    t = pick_quad(tg_ref, 4, 5)
    c = pick_quad(tcg_ref, 6, 7)

    ones_row = jnp.ones((1, DIM), jnp.float32)
    hh = _row_dots(h, h, ones_row)
    tt = _row_dots(t, t, ones_row)
    cc = _row_dots(c, c, ones_row)
    rr = _row_dots(r, r, ones_row)
    hr = _row_dots(h, r, ones_row)
    ht = _row_dots(h, t, ones_row)
    hc = _row_dots(h, c, ones_row)
    rt = _row_dots(r, t, ones_row)
    rc = _row_dots(r, c, ones_row)
    dh = jnp.maximum(jnp.sqrt(hh), 1e-12)
    dt = jnp.maximum(jnp.sqrt(tt), 1e-12)
    dc = jnp.maximum(jnp.sqrt(cc), 1e-12)
    nh = hh / (dh * dh)
    pos2 = (nh + tt / (dt * dt) + rr
            + 2.0 * (hr / dh - ht / (dh * dt) - rt / dt))
    neg2 = (nh + cc / (dc * dc) + rr
            + 2.0 * (hr / dh - hc / (dh * dc) - rc / dc))
    ps = -jnp.sqrt(jnp.maximum(pos2, 0.0) + 1e-12)
    ns = -jnp.sqrt(jnp.maximum(neg2, 0.0) + 1e-12)
    hinge = jnp.maximum(0.0, ns - ps + MARGIN)
    acc_ref[0] += jnp.sum(hinge)

    @pl.when(i == TRIP_GRID - 1)
    def _fin():
        if final:
            out_ref[0] = (acc_ref[0] / B
                          + REG_W * (stats_ref[0] + stats_ref[1]))
        else:
            out_ref[0] = acc_ref[0]


def _score(half, final, hg, rg, tg, tcg, par, stats, prev):
    gspec = pl.BlockSpec((TRIP_BLK, 2 * DIM), lambda i: (i, 0))
    off = half * TRIP_GRID
    return pl.pallas_call(
        functools.partial(_score_body, final=final),
        grid=(TRIP_GRID,),
        in_specs=[
            gspec,
            pl.BlockSpec((TRIP_BLK, 2 * DIM), lambda i: (i + off, 0)),
            gspec, gspec,
            pl.BlockSpec((8, TRIP_BLK), lambda i: (0, i + off)),
            pl.BlockSpec(memory_space=pltpu.SMEM),
            pl.BlockSpec(memory_space=pltpu.SMEM),
        ],
        out_specs=pl.BlockSpec(memory_space=pltpu.SMEM),
        out_shape=jax.ShapeDtypeStruct((1,), jnp.float32),
        scratch_shapes=[pltpu.SMEM((1,), jnp.float32)],
    )(hg, rg, tg, tcg, par, stats, prev)


TAIL_C0 = NUM_ENT - ENT_BLK
TAIL_Q0 = NUM_QUAD - QUARTER


def _ent_quad_idx(e):
    off = e % ENT_BLK
    std_q = (e // ENT_BLK) * QUARTER + (off % QUARTER)
    std_k = off // QUARTER
    toff = e - TAIL_C0
    tail_q = TAIL_Q0 + (toff % QUARTER)
    tail_k = toff // QUARTER
    in_std = e < TAIL_C0
    q = jnp.where(in_std, std_q, tail_q)
    k = jnp.where(in_std, std_k, tail_k)
    return q, k >> 1, k & 1


def kernel(entity_table, relation_table, triplets, corrupted_triplets):
    heads = triplets[:, 0]
    rels = triplets[:, 1]
    tails = triplets[:, 2]
    ctails = corrupted_triplets[:, 2]

    ent_t = entity_table.T
    rel_pair = relation_table.reshape(NUM_REL // 2, 2 * DIM)

    rg = _sc_rel()(rel_pair, rels >> 1)

    quad_table, stats = _scan_repack(ent_t, rel_pair)

    hq, hh, hw = _ent_quad_idx(heads)
    tq_, th, tw = _ent_quad_idx(tails)
    cq, ch, cw = _ent_quad_idx(ctails)

    zero = jnp.zeros((B,), jnp.int32)
    par = jnp.stack([hh, hw, rels & 1, zero, th, tw, ch, cw],
                    axis=0).astype(jnp.float32)

    hg0, tg0, cg0 = _sc_ent()(quad_table, hq[:HALF_B], tq_[:HALF_B],
                              cq[:HALF_B])
    hg1, tg1, cg1 = _sc_ent()(quad_table, hq[HALF_B:], tq_[HALF_B:],
                              cq[HALF_B:])

    part = _score(0, False, hg0, rg, tg0, cg0, par, stats,
                  jnp.zeros((1,), jnp.float32))
    out = _score(1, True, hg1, rg, tg1, cg1, par, stats, part)
    return out[0]
